# baseline (speedup 1.0000x reference)
import jax
import jax.numpy as jnp
from jax.experimental import pallas as pl
from jax.experimental.pallas import tpu as pltpu

K = 4


def _encoder_kernel(adj_ref, x_ref, lin0_w_ref, lin0_b_ref, gin_w_ref,
                    gin_b_ref, w_ih_ref, w_hh_ref, b_ih_ref, b_hh_ref,
                    out_ref, adj_sc, out0_sc, out0bf_sc, agg_sc):
    f32 = jnp.float32
    bf16 = jnp.bfloat16
    k = pl.program_id(1)
    nu = adj_ref.shape[1]
    chunk = adj_ref[0].astype(bf16)
    adj_sc[pl.ds(k * nu, nu), :] = chunk
    H = out0_sc.shape[0]

    @pl.when(k == 0)
    def _init():
        x = x_ref[0]
        out_t = jax.nn.relu(
            jax.lax.dot_general(lin0_w_ref[...], x,
                                (((1,), (1,)), ((), ())),
                                preferred_element_type=f32)
            + lin0_b_ref[...])
        out0_sc[...] = out_t
        out0bf_sc[...] = out_t.astype(bf16)
        agg_sc[...] = jnp.zeros_like(agg_sc)

    agg_sc[...] += jax.lax.dot_general(
        out0bf_sc[:, pl.ds(k * nu, nu)], chunk,
        (((1,), (0,)), ((), ())), preferred_element_type=f32)

    @pl.when(k == K - 1)
    def _finish():
        def gru(m_t, h_t):
            gi = jax.lax.dot_general(w_ih_ref[...], m_t,
                                     (((1,), (0,)), ((), ())),
                                     preferred_element_type=f32) + b_ih_ref[...]
            gh = jax.lax.dot_general(w_hh_ref[...], h_t,
                                     (((1,), (0,)), ((), ())),
                                     preferred_element_type=f32) + b_hh_ref[...]
            r = jax.nn.sigmoid(gi[:H] + gh[:H])
            z = jax.nn.sigmoid(gi[H:2 * H] + gh[H:2 * H])
            n = jnp.tanh(gi[2 * H:] + r * gh[2 * H:])
            return (1.0 - z) * n + z * h_t

        def gin(x_t):
            return jax.nn.relu(
                jax.lax.dot_general(gin_w_ref[...], x_t,
                                    (((1,), (0,)), ((), ())),
                                    preferred_element_type=f32)
                + gin_b_ref[...])

        out0 = out0_sc[...]
        out1 = gru(gin(out0 + agg_sc[...]), out0)
        agg2 = jax.lax.dot_general(out1.astype(bf16), adj_sc[...],
                                   (((1,), (0,)), ((), ())),
                                   preferred_element_type=f32)
        out2 = gru(gin(out1 + agg2), out1)
        out_ref[0] = out2


def kernel(adj, n_feat, lin0_w, lin0_b, gin_w, gin_b, gru_w_ih, gru_w_hh,
           gru_b_ih, gru_b_hh):
    B, N, FT = n_feat.shape
    H = lin0_w.shape[0]

    full = lambda shape: pl.BlockSpec(shape, lambda b, k: (0,) * len(shape))
    out3 = pl.pallas_call(
        _encoder_kernel,
        grid=(B, K),
        in_specs=[
            pl.BlockSpec((1, N // K, N), lambda b, k: (b, k, 0)),
            pl.BlockSpec((1, N, FT), lambda b, k: (b, 0, 0)),
            full((H, FT)),
            full((H, 1)),
            full((H, H)),
            full((H, 1)),
            full((3 * H, H)),
            full((3 * H, H)),
            full((3 * H, 1)),
            full((3 * H, 1)),
        ],
        out_specs=pl.BlockSpec((1, H, N), lambda b, k: (b, 0, 0)),
        out_shape=jax.ShapeDtypeStruct((B, H, N), jnp.float32),
        scratch_shapes=[
            pltpu.VMEM((N, N), jnp.bfloat16),
            pltpu.VMEM((H, N), jnp.float32),
            pltpu.VMEM((H, N), jnp.bfloat16),
            pltpu.VMEM((H, N), jnp.float32),
        ],
        compiler_params=pltpu.CompilerParams(
            dimension_semantics=("arbitrary", "arbitrary")),
    )(adj, n_feat, lin0_w, lin0_b.reshape(H, 1), gin_w, gin_b.reshape(H, 1),
      gru_w_ih, gru_w_hh, gru_b_ih.reshape(3 * H, 1),
      gru_b_hh.reshape(3 * H, 1))
    return out3.transpose(0, 2, 1).reshape(B * N, H)


# speedup vs baseline: 1.3111x; 1.3111x over previous
import jax
import jax.numpy as jnp
from jax.experimental import pallas as pl
from jax.experimental.pallas import tpu as pltpu

STEPS = 2


def _encoder_kernel(adj_ref, x_ref, lin0_w_ref, lin0_b_ref, gin_w_ref,
                    gin_b_ref, w_ih_ref, w_hh_ref, b_ih_ref, b_hh_ref,
                    out_ref):
    f32 = jnp.float32
    bf16 = jnp.bfloat16
    adj_b = adj_ref[0].astype(bf16)
    x = x_ref[0]

    out_t = jax.nn.relu(
        jax.lax.dot_general(lin0_w_ref[...], x,
                            (((1,), (1,)), ((), ())),
                            preferred_element_type=f32)
        + lin0_b_ref[...])
    h_t = out_t
    H = out_t.shape[0]

    for _ in range(STEPS):
        agg_t = jax.lax.dot_general(out_t.astype(bf16), adj_b,
                                    (((1,), (0,)), ((), ())),
                                    preferred_element_type=f32)
        m_t = jax.nn.relu(
            jax.lax.dot_general(gin_w_ref[...], out_t + agg_t,
                                (((1,), (0,)), ((), ())),
                                preferred_element_type=f32)
            + gin_b_ref[...])
        gi = jax.lax.dot_general(w_ih_ref[...], m_t,
                                 (((1,), (0,)), ((), ())),
                                 preferred_element_type=f32) + b_ih_ref[...]
        gh = jax.lax.dot_general(w_hh_ref[...], h_t,
                                 (((1,), (0,)), ((), ())),
                                 preferred_element_type=f32) + b_hh_ref[...]
        r = jax.nn.sigmoid(gi[:H] + gh[:H])
        z = jax.nn.sigmoid(gi[H:2 * H] + gh[H:2 * H])
        n = jnp.tanh(gi[2 * H:] + r * gh[2 * H:])
        out_t = (1.0 - z) * n + z * h_t
        h_t = out_t

    out_ref[0] = out_t


def kernel(adj, n_feat, lin0_w, lin0_b, gin_w, gin_b, gru_w_ih, gru_w_hh,
           gru_b_ih, gru_b_hh):
    B, N, FT = n_feat.shape
    H = lin0_w.shape[0]

    full = lambda shape: pl.BlockSpec(shape, lambda b: (0,) * len(shape))
    out3 = pl.pallas_call(
        _encoder_kernel,
        grid=(B,),
        in_specs=[
            pl.BlockSpec((1, N, N), lambda b: (b, 0, 0)),
            pl.BlockSpec((1, N, FT), lambda b: (b, 0, 0)),
            full((H, FT)),
            full((H, 1)),
            full((H, H)),
            full((H, 1)),
            full((3 * H, H)),
            full((3 * H, H)),
            full((3 * H, 1)),
            full((3 * H, 1)),
        ],
        out_specs=pl.BlockSpec((1, H, N), lambda b: (b, 0, 0)),
        out_shape=jax.ShapeDtypeStruct((B, H, N), jnp.float32),
        compiler_params=pltpu.CompilerParams(
            dimension_semantics=("parallel",)),
    )(adj, n_feat, lin0_w, lin0_b.reshape(H, 1), gin_w, gin_b.reshape(H, 1),
      gru_w_ih, gru_w_hh, gru_b_ih.reshape(3 * H, 1),
      gru_b_hh.reshape(3 * H, 1))
    return out3.transpose(0, 2, 1).reshape(B * N, H)
